# Initial kernel scaffold; baseline (speedup 1.0000x reference)
import jax
import jax.numpy as jnp
from jax.experimental import pallas as pl


def kernel(node_features, edge_index, edge_features, is_leaf, params):
    raise NotImplementedError("write your pallas kernel here")


# speedup vs baseline: 36.0147x; 36.0147x over previous
import jax
import jax.numpy as jnp
import numpy as np
from jax.experimental import pallas as pl

N = 100000
NP = 131072
LEVELS = 16
B = 2048
NB = NP // B
F32 = jnp.float32


def _pairsum(a):
    r = a.shape[0]
    return a.reshape(r // 2, 2, 16).sum(axis=1)


def _accin(v):
    lf = v[:, 13:14]
    return jnp.concatenate(
        [v[:, 0:13] * lf, lf, jnp.zeros((v.shape[0], 2), F32)], axis=1)


def _finalize(v, acc):
    lf = v[:, 13:14]
    cnt = acc[:, 13:14]
    avg = acc[:, 0:13] / jnp.maximum(cnt, 1.0)
    fill = (lf == 0.0) & (cnt > 0.0)
    res = jnp.where(fill, avg, v[:, 0:13])
    return jnp.concatenate([res, jnp.zeros((v.shape[0], 3), F32)], axis=1)


def _deep_kernel(i13_ref, i14_ref, i15_ref, i16_ref,
                 f13_ref, f14_ref, f15_ref, f16_ref, a13_ref):
    v16 = i16_ref[...]
    a16 = _accin(v16)
    v15 = i15_ref[...]
    a15 = _accin(v15) + _pairsum(a16)
    v14 = i14_ref[...]
    a14 = _accin(v14) + _pairsum(a15)
    v13 = i13_ref[...]
    a13 = _accin(v13) + _pairsum(a14)
    a13_ref[...] = a13
    f13_ref[...] = _finalize(v13, a13)
    f14_ref[...] = _finalize(v14, a14)
    f15_ref[...] = _finalize(v15, a15)
    f16_ref[...] = _finalize(v16, a16)


def _top_kernel(init_ref, a13_ref, out_ref):
    v = init_ref[...]
    out_ref[...] = _accin(v)
    ps13 = _pairsum(a13_ref[...])
    out_ref[4096:8192, :] = out_ref[4096:8192, :] + ps13
    for d in range(12, 3, -1):
        plo = 1 << (d - 1)
        ps = _pairsum(out_ref[2 * plo:4 * plo, :])
        out_ref[plo:2 * plo, :] = out_ref[plo:2 * plo, :] + ps
    t = out_ref[0:16, :]
    for d in (3, 2, 1):
        clo = 1 << d
        half = clo // 2
        ps = t[clo:2 * clo, :].reshape(half, 2, 16).sum(axis=1)
        upd = jnp.concatenate(
            [jnp.zeros((half, 16), F32), ps,
             jnp.zeros((16 - 2 * half, 16), F32)], axis=0)
        t = t + upd
    out_ref[0:16, :] = t
    out_ref[...] = _finalize(v, out_ref[...])


def _proj_kernel(x_ref, w1_ref, b1_ref, w2_ref, b2_ref, o_ref):
    h = jnp.dot(x_ref[...], w1_ref[...], preferred_element_type=F32) + b1_ref[...]
    h = jnp.maximum(h, 0.0)
    o_ref[...] = jnp.dot(h, w2_ref[...], preferred_element_type=F32) + b2_ref[...]


def _rep2(a, rows, cols):
    return jnp.broadcast_to(a[:, None, :], (rows, 2, cols)).reshape(2 * rows, cols)


def _lrelu(z):
    return jnp.where(z >= 0.0, z, 0.2 * z)


def _gat_kernel(xs_ref, xp_ref, c0_ref, c1_ref, w_ref, a_ref, b_ref,
                g_ref, lb_ref, o_ref):
    b = pl.program_id(0)
    xs = xs_ref[...]
    off = (b % 2) * (B // 2)
    ph = xp_ref[pl.ds(off, B // 2), :]
    C = jnp.concatenate([c0_ref[...], c1_ref[...]], axis=0)
    w = w_ref[...]
    A = a_ref[...]

    hs = jnp.dot(xs, w, preferred_element_type=F32)
    hph = jnp.dot(ph, w, preferred_element_type=F32)
    hC = jnp.dot(C, w, preferred_element_type=F32)

    ab_s = jnp.dot(hs, A, preferred_element_type=F32)
    as_p = jnp.dot(hph, A[:, 0:4], preferred_element_type=F32)
    as_C = jnp.dot(hC, A[:, 0:4], preferred_element_type=F32)

    hp = _rep2(hph, B // 2, 256)
    ap = _rep2(as_p, B // 2, 4)
    hC3 = hC.reshape(B, 2, 256)
    h1, h2 = hC3[:, 0, :], hC3[:, 1, :]
    aC3 = as_C.reshape(B, 2, 4)
    a1, a2 = aC3[:, 0, :], aC3[:, 1, :]

    r = b * B + jax.lax.broadcasted_iota(jnp.int32, (B, 1), 0)
    has_p = r >= 2
    has_1 = (2 * r) <= N
    has_2 = (2 * r + 1) <= N

    adst = ab_s[:, 4:8]
    e_s = _lrelu(ab_s[:, 0:4] + adst)
    e_p = _lrelu(ap + adst)
    e_1 = _lrelu(a1 + adst)
    e_2 = _lrelu(a2 + adst)

    NEG = jnp.float32(-1e30)
    m = jnp.maximum(e_s, jnp.where(has_p, e_p, NEG))
    m = jnp.maximum(m, jnp.where(has_1, e_1, NEG))
    m = jnp.maximum(m, jnp.where(has_2, e_2, NEG))

    w_s = jnp.exp(e_s - m)
    w_p = jnp.where(has_p, jnp.exp(e_p - m), 0.0)
    w_1 = jnp.where(has_1, jnp.exp(e_1 - m), 0.0)
    w_2 = jnp.where(has_2, jnp.exp(e_2 - m), 0.0)
    s = w_s + w_p + w_1 + w_2 + 1e-16

    lane = jax.lax.broadcasted_iota(jnp.int32, (4, 256), 1)
    row = jax.lax.broadcasted_iota(jnp.int32, (4, 256), 0)
    E = (lane // 64 == row).astype(F32)

    def ex(a):
        return jnp.dot(a / s, E, preferred_element_type=F32)

    out_flat = ex(w_s) * hs + ex(w_p) * hp + ex(w_1) * h1 + ex(w_2) * h2

    mi = jax.lax.broadcasted_iota(jnp.int32, (256, 64), 0)
    mj = jax.lax.broadcasted_iota(jnp.int32, (256, 64), 1)
    M = jnp.where(mi % 64 == mj, jnp.float32(0.25), 0.0)
    out64 = jnp.dot(out_flat, M, preferred_element_type=F32) + b_ref[...]

    y = xs + out64
    mu = jnp.mean(y, axis=1, keepdims=True)
    var = jnp.mean((y - mu) ** 2, axis=1, keepdims=True)
    o_ref[...] = (y - mu) * jax.lax.rsqrt(var + 1e-5) * g_ref[...] + lb_ref[...]


def _edge_kernel(xc_ref, xp_ref, ef_ref, w1p_ref, w1c_ref, w1e_ref, b1_ref,
                 w2_ref, b2_ref, hw1_ref, hb1_ref, hw2_ref, hb2_ref,
                 emb_ref, log_ref):
    b = pl.program_id(0)
    xc = xc_ref[...]
    off = (b % 2) * (B // 2)
    ph = xp_ref[pl.ds(off, B // 2), :]
    xp = _rep2(ph, B // 2, 64)
    z = (jnp.dot(xp, w1p_ref[...], preferred_element_type=F32)
         + jnp.dot(xc, w1c_ref[...], preferred_element_type=F32)
         + jnp.dot(ef_ref[...], w1e_ref[...], preferred_element_type=F32)
         + b1_ref[...])
    emb = jnp.dot(jnp.maximum(z, 0.0), w2_ref[...],
                  preferred_element_type=F32) + b2_ref[...]
    emb_ref[...] = emb
    l1 = jnp.maximum(jnp.dot(emb, hw1_ref[...], preferred_element_type=F32)
                     + hb1_ref[...], 0.0)
    log_ref[...] = jnp.dot(l1, hw2_ref[...], preferred_element_type=F32) + hb2_ref[...]


def _full(shape):
    return pl.BlockSpec(shape, lambda b: (0, 0))


def kernel(node_features, edge_index, edge_features, is_leaf, params):
    del edge_index
    nf = node_features.astype(F32)
    leaf = is_leaf.astype(F32)[:, None]
    init = jnp.pad(jnp.concatenate([nf, leaf], axis=1),
                   ((1, NP - 1 - N), (0, 2)))

    G = 8
    f13, f14, f15, f16, acc13 = pl.pallas_call(
        _deep_kernel,
        grid=(G,),
        in_specs=[pl.BlockSpec((1024, 16), lambda g: (8 + g, 0)),
                  pl.BlockSpec((2048, 16), lambda g: (8 + g, 0)),
                  pl.BlockSpec((4096, 16), lambda g: (8 + g, 0)),
                  pl.BlockSpec((8192, 16), lambda g: (8 + g, 0))],
        out_specs=[pl.BlockSpec((1024, 16), lambda g: (g, 0)),
                   pl.BlockSpec((2048, 16), lambda g: (g, 0)),
                   pl.BlockSpec((4096, 16), lambda g: (g, 0)),
                   pl.BlockSpec((8192, 16), lambda g: (g, 0)),
                   pl.BlockSpec((1024, 16), lambda g: (g, 0))],
        out_shape=[jax.ShapeDtypeStruct((8192, 16), F32),
                   jax.ShapeDtypeStruct((16384, 16), F32),
                   jax.ShapeDtypeStruct((32768, 16), F32),
                   jax.ShapeDtypeStruct((65536, 16), F32),
                   jax.ShapeDtypeStruct((8192, 16), F32)],
    )(init, init, init, init)

    top = pl.pallas_call(
        _top_kernel,
        grid=(1,),
        in_specs=[pl.BlockSpec((8192, 16), lambda i: (0, 0)),
                  pl.BlockSpec((8192, 16), lambda i: (0, 0))],
        out_specs=pl.BlockSpec((8192, 16), lambda i: (0, 0)),
        out_shape=jax.ShapeDtypeStruct((8192, 16), F32),
    )(init, acc13)

    x16 = jnp.concatenate([top, f13, f14, f15, f16], axis=0)

    w1p = jnp.pad(params['proj_w1'], ((0, 3), (0, 0)))
    x = pl.pallas_call(
        _proj_kernel,
        grid=(NB,),
        in_specs=[pl.BlockSpec((B, 16), lambda b: (b, 0)),
                  _full((16, 64)), _full((1, 64)),
                  _full((64, 64)), _full((1, 64))],
        out_specs=pl.BlockSpec((B, 64), lambda b: (b, 0)),
        out_shape=jax.ShapeDtypeStruct((NP, 64), F32),
    )(x16, w1p, params['proj_b1'][None, :],
      params['proj_w2'], params['proj_b2'][None, :])

    for p in params['gat']:
        att = jnp.zeros((256, 8), F32)
        for h in range(4):
            att = att.at[h * 64:(h + 1) * 64, h].set(p['att_src'][h])
            att = att.at[h * 64:(h + 1) * 64, 4 + h].set(p['att_dst'][h])
        x = pl.pallas_call(
            _gat_kernel,
            grid=(NB,),
            in_specs=[pl.BlockSpec((B, 64), lambda b: (b, 0)),
                      pl.BlockSpec((B, 64), lambda b: (b // 2, 0)),
                      pl.BlockSpec((B, 64),
                                   lambda b: (jnp.minimum(2 * b, NB - 1), 0)),
                      pl.BlockSpec((B, 64),
                                   lambda b: (jnp.minimum(2 * b + 1, NB - 1), 0)),
                      _full((64, 256)), _full((256, 8)), _full((1, 64)),
                      _full((1, 64)), _full((1, 64))],
            out_specs=pl.BlockSpec((B, 64), lambda b: (b, 0)),
            out_shape=jax.ShapeDtypeStruct((NP, 64), F32),
        )(x, x, x, x, p['w'], att, p['bias'][None, :],
          p['ln_g'][None, :], p['ln_b'][None, :])

    ef_pad = jnp.pad(edge_features.astype(F32), ((2, NP - 2 - (N - 1)), (0, 0)))
    emb_pad, log_pad = pl.pallas_call(
        _edge_kernel,
        grid=(NB,),
        in_specs=[pl.BlockSpec((B, 64), lambda b: (b, 0)),
                  pl.BlockSpec((B, 64), lambda b: (b // 2, 0)),
                  pl.BlockSpec((B, 4), lambda b: (b, 0)),
                  _full((64, 64)), _full((64, 64)), _full((4, 64)),
                  _full((1, 64)), _full((64, 64)), _full((1, 64)),
                  _full((64, 64)), _full((1, 64)), _full((64, 2)),
                  _full((1, 2))],
        out_specs=[pl.BlockSpec((B, 64), lambda b: (b, 0)),
                   pl.BlockSpec((B, 2), lambda b: (b, 0))],
        out_shape=[jax.ShapeDtypeStruct((NP, 64), F32),
                   jax.ShapeDtypeStruct((NP, 2), F32)],
    )(x, x, ef_pad,
      params['emlp_w1'][0:64], params['emlp_w1'][64:128], params['emlp_w1'][128:132],
      params['emlp_b1'][None, :], params['emlp_w2'], params['emlp_b2'][None, :],
      params['head_w1'], params['head_b1'][None, :],
      params['head_w2'], params['head_b2'][None, :])

    return log_pad[2:N + 1], emb_pad[2:N + 1]
